# Initial kernel scaffold; baseline (speedup 1.0000x reference)
import jax
import jax.numpy as jnp
from jax.experimental import pallas as pl


def kernel(graph_x, graph_edge_index, boundary_x, boundary_edge_index, g1_W, g1_as, g1_ad, g1_b, g2_W, g2_as, g2_ad, g2_b, g3_W, g3_as, g3_ad, g3_b, g4_W, g4_as, g4_ad, g4_b, b1_W, b1_as, b1_ad, b1_b, b2_W, b2_as, b2_ad, b2_b, c1_W, c1_as, c1_ad, c1_b, wl1_W, wl1_b, wo_W, wo_b, hl1_W, hl1_b, ho_W, ho_b):
    raise NotImplementedError("write your pallas kernel here")


# speedup vs baseline: 16.1904x; 16.1904x over previous
import functools

import jax
import jax.numpy as jnp
from jax import lax
from jax.experimental import pallas as pl
from jax.experimental.pallas import tpu as pltpu
from jax.experimental.pallas import tpu_sc as plsc


def _lr(v):
    return jnp.where(v > 0, v, 0.01 * v)


def _lr2(v):
    return jnp.where(v > 0, v, 0.2 * v)


def _mm_body(x_ref, w_ref, o_ref):
    o_ref[...] = jnp.dot(x_ref[...], w_ref[...],
                         preferred_element_type=jnp.float32)


@functools.partial(jax.jit, static_argnames=("bm",))
def _mm(x, w, bm=2000):
    m, k = x.shape
    _, n = w.shape
    return pl.pallas_call(
        _mm_body,
        grid=(m // bm,),
        in_specs=[
            pl.BlockSpec((bm, k), lambda i: (i, 0)),
            pl.BlockSpec((k, n), lambda i: (0, 0)),
        ],
        out_specs=pl.BlockSpec((bm, n), lambda i: (i, 0)),
        out_shape=jax.ShapeDtypeStruct((m, n), jnp.float32),
    )(x, w)


_CB = 128
_CA = 512
_NTILES = 16


def _dyn_bcast(vec, idx16):
    return lax.gather(
        vec, idx16[:, None],
        dimension_numbers=lax.GatherDimensionNumbers(
            offset_dims=(), collapsed_slice_dims=(0,), start_index_map=(0,)),
        slice_sizes=(1,),
        mode=lax.GatherScatterMode.PROMISE_IN_BOUNDS)


def _sc_edge_call(alsT, aldT, srcp, dstp, hS, kvec, NP, H, C, EP):
    S = (H * C) // 128
    HPS = 128 // C
    S2 = S // 2
    spc = max(S2, 1)
    nheads_a = spc * HPS
    nca = EP // _NTILES // _CA
    ncb = EP // _NTILES // _CB
    stripe = NP // _NTILES
    nb = stripe // _CB
    mesh = plsc.VectorSubcoreMesh(core_axis_name="c", subcore_axis_name="s")

    @functools.partial(
        pl.kernel, mesh=mesh,
        compiler_params=pltpu.CompilerParams(needs_layout_passes=False),
        out_type=[
            jax.ShapeDtypeStruct((S * NP, 128), jnp.float32),
            jax.ShapeDtypeStruct((16, EP), jnp.float32),
        ],
        scratch_types=[
            pltpu.VMEM((8, stripe), jnp.float32),
            pltpu.VMEM((16,), jnp.float32),
            pltpu.VMEM_SHARED((_NTILES, NP), jnp.float32),
            pltpu.VMEM_SHARED((NP, 128), jnp.float32),
            pltpu.SemaphoreType.DMA,
        ],
    )
    def k(alsT_h, aldT_h, src_h, dst_h, hS_h, kv_h, out_h, w_h,
          sstripe_all, kvv, s_parts, acc, sem):
        c = lax.axis_index("c")
        sid = lax.axis_index("s")
        sbase = sid * stripe
        z16 = jnp.zeros((16,), jnp.float32)

        pltpu.sync_copy(kv_h, kvv)

        def stage_a(als_v, ald_v, s_v, ia_s, ia_d, wcol, spart_v):
            for ja in range(nheads_a):
                h = c * (S2 * HPS) + ja
                pltpu.sync_copy(alsT_h.at[h], als_v)
                pltpu.sync_copy(aldT_h.at[h], ald_v)
                kb = _dyn_bcast(kvv[:], jnp.broadcast_to(h, (16,)))

                def zs(i, _):
                    s_v[pl.ds(i * 16, 16)] = z16
                    return 0
                lax.fori_loop(0, NP // 16, zs, 0)

                def achunk(ch, _):
                    eb = (sid * nca + ch) * _CA
                    pltpu.sync_copy(src_h.at[pl.ds(eb, _CA)], ia_s)
                    pltpu.sync_copy(dst_h.at[pl.ds(eb, _CA)], ia_d)

                    def agrp(g, _):
                        sl = pl.ds(g * 16, 16)
                        s16 = ia_s[sl]
                        d16 = ia_d[sl]
                        a = plsc.load_gather(als_v, [s16])
                        d = plsc.load_gather(ald_v, [d16])
                        v = a + d
                        v = jnp.where(v > 0, v, 0.2 * v)
                        w16 = jnp.exp(v - kb)
                        wcol[sl] = w16
                        plsc.addupdate_scatter(s_v, [d16], w16)
                        return 0
                    lax.fori_loop(0, _CA // 16, agrp, 0)
                    pltpu.sync_copy(wcol, w_h.at[h, pl.ds(eb, _CA)])
                    return 0
                lax.fori_loop(0, nca, achunk, 0)

                plsc.subcore_barrier()
                pltpu.sync_copy(s_v, s_parts.at[sid])
                plsc.subcore_barrier()
                for p in range(_NTILES):
                    pltpu.sync_copy(
                        s_parts.at[p, pl.ds(sbase, stripe)], spart_v)
                    if p == 0:
                        def red0(i, _):
                            sl = pl.ds(i * 16, 16)
                            sstripe_all[ja, sl] = spart_v[sl]
                            return 0
                        lax.fori_loop(0, stripe // 16, red0, 0)
                    else:
                        def red(i, _):
                            sl = pl.ds(i * 16, 16)
                            sstripe_all[ja, sl] = (
                                sstripe_all[ja, sl] + spart_v[sl])
                            return 0
                        lax.fori_loop(0, stripe // 16, red, 0)
                plsc.subcore_barrier()

        pl.run_scoped(
            stage_a,
            pltpu.VMEM((NP,), jnp.float32),
            pltpu.VMEM((NP,), jnp.float32),
            pltpu.VMEM((NP,), jnp.float32),
            pltpu.VMEM((_CA,), jnp.int32),
            pltpu.VMEM((_CA,), jnp.int32),
            pltpu.VMEM((_CA,), jnp.float32),
            pltpu.VMEM((stripe,), jnp.float32),
        )

        def stage_b(idx_sb, idx_db, gidx, whbuf, rows):
            def zrow(i, _):
                for jj in range(8):
                    rows[i, pl.ds(jj * 16, 16)] = z16
                return 0
            lax.fori_loop(0, _CB, zrow, 0)
            for b in range(nb):
                pltpu.sync_copy(rows, acc.at[pl.ds(sbase + b * _CB, _CB)])
            plsc.subcore_barrier()

            for js in range(spc):
                g = c * S2 + js

                def bchunk(ch, _):
                    eb = (sid * ncb + ch) * _CB
                    pltpu.sync_copy(src_h.at[pl.ds(eb, _CB)], idx_sb)
                    pltpu.sync_copy(dst_h.at[pl.ds(eb, _CB)], idx_db)
                    for jj in range(8):
                        sl = pl.ds(jj * 16, 16)
                        gidx[sl] = idx_sb[sl] + g * NP
                    pltpu.async_copy(hS_h.at[gidx], rows, sem).wait()
                    for kk in range(HPS):
                        pltpu.sync_copy(
                            w_h.at[g * HPS + kk, pl.ds(eb, _CB)],
                            whbuf.at[kk])

                    def sbody(i, _):
                        iv = jnp.broadcast_to(i & 15, (16,))
                        ib = (i >> 4) << 4
                        avs = []
                        for kk in range(HPS):
                            wvec = whbuf[kk, pl.ds(ib, 16)]
                            avs.append(_dyn_bcast(wvec, iv))
                        for jj in range(8):
                            sl = pl.ds(jj * 16, 16)
                            rows[i, sl] = rows[i, sl] * avs[(jj * 16) // C]
                        return 0
                    lax.fori_loop(0, _CB, sbody, 0)
                    pltpu.sync_copy(rows, acc.at[idx_db], add=True)
                    return 0
                lax.fori_loop(0, ncb, bchunk, 0)
                plsc.subcore_barrier()

                for b in range(nb):
                    pltpu.sync_copy(
                        acc.at[pl.ds(sbase + b * _CB, _CB)], rows)

                    def nbody(i, _):
                        iv = jnp.broadcast_to(i & 15, (16,))
                        nb16 = b * _CB + ((i >> 4) << 4)
                        svs = []
                        for kk in range(HPS):
                            svvec = sstripe_all[
                                js * HPS + kk, pl.ds(nb16, 16)]
                            svs.append(_dyn_bcast(svvec, iv) + 1e-16)
                        for jj in range(8):
                            sl = pl.ds(jj * 16, 16)
                            rows[i, sl] = rows[i, sl] / svs[(jj * 16) // C]
                        return 0
                    lax.fori_loop(0, _CB, nbody, 0)
                    pltpu.sync_copy(
                        rows,
                        out_h.at[pl.ds(g * NP + sbase + b * _CB, _CB)])
                    lax.fori_loop(0, _CB, zrow, 0)
                    pltpu.sync_copy(
                        rows, acc.at[pl.ds(sbase + b * _CB, _CB)])
                plsc.subcore_barrier()

        pl.run_scoped(
            stage_b,
            pltpu.VMEM((_CB,), jnp.int32),
            pltpu.VMEM((_CB,), jnp.int32),
            pltpu.VMEM((_CB,), jnp.int32),
            pltpu.VMEM((4, _CB), jnp.float32),
            pltpu.VMEM((_CB, 128), jnp.float32),
        )

    return k(alsT, aldT, srcp, dstp, hS, kvec)


def _pad_edges(ei, n):
    e = ei.shape[1]
    ar = jnp.arange(n, dtype=ei.dtype)
    src = jnp.concatenate([ei[0], ar])
    dst = jnp.concatenate([ei[1], ar])
    ep = -(-(e + n) // (_NTILES * _CA)) * (_NTILES * _CA)
    pad = ep - (e + n)
    fill = jnp.full((pad,), n, ei.dtype)
    return jnp.concatenate([src, fill]), jnp.concatenate([dst, fill]), ep


def _gat_sc(x, srcp, dstp, ep, W, a_s, a_d, b, np_):
    n = x.shape[0]
    h_heads, c = a_s.shape
    s = (h_heads * c) // 128
    h = _mm(x, W)
    h3 = h.reshape(n, h_heads, c)
    als = jnp.sum(h3 * a_s[None], axis=-1)
    ald = jnp.sum(h3 * a_d[None], axis=-1)
    kv = _lr2(jnp.max(als, 0) + jnp.max(ald, 0))
    kvec = jnp.concatenate([kv, jnp.zeros((16 - h_heads,), jnp.float32)])
    alsT = jnp.pad(als.T, ((0, 16 - h_heads), (0, np_ - n)))
    aldT = jnp.pad(ald.T, ((0, 16 - h_heads), (0, np_ - n)))
    hS = jnp.pad(h.reshape(n, s, 128).transpose(1, 0, 2),
                 ((0, 0), (0, np_ - n), (0, 0))).reshape(s * np_, 128)
    out, _ = _sc_edge_call(alsT, aldT, srcp, dstp, hS, kvec,
                           np_, h_heads, c, ep)
    out = out.reshape(s, np_, 128)[:, :n]
    return out.transpose(1, 0, 2).reshape(n, h_heads * c) + b


def kernel(graph_x, graph_edge_index, boundary_x, boundary_edge_index, g1_W, g1_as, g1_ad, g1_b, g2_W, g2_as, g2_ad, g2_b, g3_W, g3_as, g3_ad, g3_b, g4_W, g4_as, g4_ad, g4_b, b1_W, b1_as, b1_ad, b1_b, b2_W, b2_as, b2_ad, b2_b, c1_W, c1_as, c1_ad, c1_b, wl1_W, wl1_b, wo_W, wo_b, hl1_W, hl1_b, ho_W, ho_b):
    n = graph_x.shape[0]
    nbn = boundary_x.shape[0]
    np_ = -(-(n + 1) // (_NTILES * _CB)) * (_NTILES * _CB)
    npb = -(-(nbn + 1) // (_NTILES * _CB)) * (_NTILES * _CB)
    gsrc, gdst, gep = _pad_edges(graph_edge_index, n)
    bsrc, bdst, bep = _pad_edges(boundary_edge_index, nbn)

    res = graph_x
    xg = _lr(_gat_sc(graph_x, gsrc, gdst, gep, g1_W, g1_as, g1_ad, g1_b, np_))
    xg = jnp.concatenate([xg, res], axis=1)
    xg = _lr(_gat_sc(xg, gsrc, gdst, gep, g2_W, g2_as, g2_ad, g2_b, np_))
    xg = jnp.concatenate([xg, res], axis=1)
    xg = _lr(_gat_sc(xg, gsrc, gdst, gep, g3_W, g3_as, g3_ad, g3_b, np_))
    xg = jnp.concatenate([xg, res], axis=1)
    xg = _lr(_gat_sc(xg, gsrc, gdst, gep, g4_W, g4_as, g4_ad, g4_b, np_))
    xg = jnp.concatenate([xg, res], axis=1)

    bres = boundary_x
    xb = _lr(_gat_sc(boundary_x, bsrc, bdst, bep,
                     b1_W, b1_as, b1_ad, b1_b, npb))
    xb = jnp.concatenate([xb, bres], axis=1)
    xb = _lr(_gat_sc(xb, bsrc, bdst, bep, b2_W, b2_as, b2_ad, b2_b, npb))
    xb = jnp.concatenate([xb, bres], axis=1)

    pooled = jnp.max(xb, axis=0, keepdims=True)
    x = jnp.concatenate([xg, jnp.tile(pooled, (n, 1))], axis=1)
    x = _lr(_gat_sc(x, gsrc, gdst, gep, c1_W, c1_as, c1_ad, c1_b, np_))

    w = _lr(_mm(x, wl1_W) + wl1_b)
    w = w @ wo_W + wo_b
    h = _lr(_mm(x, hl1_W) + hl1_b)
    h = h @ ho_W + ho_b
    return w.squeeze(-1), h.squeeze(-1)
